# Initial kernel scaffold; baseline (speedup 1.0000x reference)
import jax
import jax.numpy as jnp
from jax.experimental import pallas as pl


def kernel(x, edge_index, W_l, b_l, W_r, W_head, b_head):
    raise NotImplementedError("write your pallas kernel here")


# speedup vs baseline: 8.6813x; 8.6813x over previous
import functools

import jax
import jax.numpy as jnp
from jax import lax
from jax.experimental import pallas as pl
from jax.experimental.pallas import tpu as pltpu
from jax.experimental.pallas import tpu_sc as plsc

N_NODES = 10000
D_IN = 128
HIDDEN = 64
N_EDGES = 320000

NC = 2
NS = 16
NW = NC * NS
CHUNK = 128
CH_PER_TILE = 79
E_PAD = NW * CH_PER_TILE * CHUNK
ROWS_PER_TILE = 632
ACC_ROWS = NS * ROWS_PER_TILE
CNT_W = 16


def _proj_body(x_ref, w_ref, z_ref):
    z_ref[...] = lax.dot_general(
        x_ref[...], w_ref[...], (((1,), (1,)), ((), ())),
        preferred_element_type=jnp.float32)


def _post_body(acc_ref, cnt_ref, x_ref, wr_ref, bl_ref, wh_ref, bh_ref, y_ref):
    agg = acc_ref[0, :N_NODES, :] + acc_ref[1, :N_NODES, :]
    cnt = cnt_ref[0, :N_NODES, 0:1] + cnt_ref[1, :N_NODES, 0:1]
    cnt = jnp.maximum(cnt, 1.0)
    xr = lax.dot_general(
        x_ref[...], wr_ref[...], (((1,), (1,)), ((), ())),
        preferred_element_type=jnp.float32)
    conv = agg / cnt + bl_ref[...] + xr
    h = jnp.maximum(conv, 0.0)
    y = lax.dot_general(
        h, wh_ref[...], (((1,), (0,)), ((), ())),
        preferred_element_type=jnp.float32)
    y_ref[...] = y + bh_ref[0, 0]


def _sc_body(z_hbm, src_hbm, dst_hbm, zrow_hbm, zcnt_hbm, ones_hbm,
             acc_out, cnt_out,
             src_v, dst_v, rows_v, ones_v, acc_sh, cnt_sh, sem):
    cid = lax.axis_index("c")
    sid = lax.axis_index("s")
    wid = sid * NC + cid
    base = sid * ROWS_PER_TILE

    pltpu.sync_copy(src_hbm.at[wid], src_v)
    pltpu.sync_copy(dst_hbm.at[wid], dst_v)
    pltpu.sync_copy(ones_hbm, ones_v)
    pltpu.sync_copy(zrow_hbm, acc_sh.at[pl.ds(base, ROWS_PER_TILE)])
    pltpu.sync_copy(zcnt_hbm, cnt_sh.at[pl.ds(base, ROWS_PER_TILE)])
    plsc.subcore_barrier()

    def body(j, carry):
        pltpu.async_copy(z_hbm.at[src_v.at[j]], rows_v, sem).wait()
        pltpu.sync_copy(rows_v, acc_sh.at[dst_v.at[j]], add=True)
        pltpu.sync_copy(ones_v, cnt_sh.at[dst_v.at[j]], add=True)
        return carry

    lax.fori_loop(0, CH_PER_TILE, body, 0)
    plsc.subcore_barrier()

    pltpu.sync_copy(acc_sh.at[pl.ds(base, ROWS_PER_TILE)],
                    acc_out.at[cid, pl.ds(base, ROWS_PER_TILE)])
    pltpu.sync_copy(cnt_sh.at[pl.ds(base, ROWS_PER_TILE)],
                    cnt_out.at[cid, pl.ds(base, ROWS_PER_TILE)])


_sc_segment_sum = functools.partial(
    pl.kernel,
    out_type=(
        jax.ShapeDtypeStruct((NC, ACC_ROWS, HIDDEN), jnp.float32),
        jax.ShapeDtypeStruct((NC, ACC_ROWS, CNT_W), jnp.float32),
    ),
    mesh=plsc.VectorSubcoreMesh(core_axis_name="c", subcore_axis_name="s"),
    compiler_params=pltpu.CompilerParams(use_tc_tiling_on_sc=False),
    scratch_types=[
        pltpu.VMEM((CH_PER_TILE, CHUNK), jnp.int32),
        pltpu.VMEM((CH_PER_TILE, CHUNK), jnp.int32),
        pltpu.VMEM((CHUNK, HIDDEN), jnp.float32),
        pltpu.VMEM((CHUNK, CNT_W), jnp.float32),
        pltpu.VMEM_SHARED((ACC_ROWS, HIDDEN), jnp.float32),
        pltpu.VMEM_SHARED((ACC_ROWS, CNT_W), jnp.float32),
        pltpu.SemaphoreType.DMA,
    ],
)(_sc_body)


@jax.jit
def kernel(x, edge_index, W_l, b_l, W_r, W_head, b_head):
    src = edge_index[0].astype(jnp.int32)
    dst = edge_index[1].astype(jnp.int32)
    pad = E_PAD - N_EDGES
    src_p = jnp.concatenate([src, jnp.zeros((pad,), jnp.int32)])
    dst_p = jnp.concatenate([dst, jnp.full((pad,), N_NODES, jnp.int32)])
    src_r = src_p.reshape(NW, CH_PER_TILE, CHUNK)
    dst_r = dst_p.reshape(NW, CH_PER_TILE, CHUNK)

    z = pl.pallas_call(
        _proj_body,
        out_shape=jax.ShapeDtypeStruct((N_NODES, HIDDEN), jnp.float32),
    )(x, W_l)

    zrow = jnp.zeros((ROWS_PER_TILE, HIDDEN), jnp.float32)
    zcnt = jnp.zeros((ROWS_PER_TILE, CNT_W), jnp.float32)
    ones = jnp.ones((CHUNK, CNT_W), jnp.float32)
    acc, cnt = _sc_segment_sum(z, src_r, dst_r, zrow, zcnt, ones)

    y = pl.pallas_call(
        _post_body,
        in_specs=[
            pl.BlockSpec(memory_space=pltpu.VMEM),
            pl.BlockSpec(memory_space=pltpu.VMEM),
            pl.BlockSpec(memory_space=pltpu.VMEM),
            pl.BlockSpec(memory_space=pltpu.VMEM),
            pl.BlockSpec(memory_space=pltpu.VMEM),
            pl.BlockSpec(memory_space=pltpu.VMEM),
            pl.BlockSpec(memory_space=pltpu.SMEM),
        ],
        out_shape=jax.ShapeDtypeStruct((N_NODES, 1), jnp.float32),
    )(acc, cnt, x, W_r, b_l.reshape(1, HIDDEN), W_head.reshape(HIDDEN, 1),
      b_head.reshape(1, 1))
    return jnp.squeeze(y, axis=-1)
